# baseline (speedup 1.0000x reference)
import functools

import jax
import jax.numpy as jnp
from jax import lax
from jax.experimental import pallas as pl
from jax.experimental.pallas import tpu as pltpu
from jax.experimental.pallas import tpu_sc as plsc

_NC = 2
_NS = 16
_NW = _NC * _NS


def _emb_gather(ids_flat, table):
    B = ids_flat.shape[0]
    D = table.shape[1]
    BW = B // _NW
    C = 8
    NBUF = 4
    SBUF = 3
    G = 3
    nchunk = BW // C

    mesh = plsc.VectorSubcoreMesh(core_axis_name="c", subcore_axis_name="s")

    @functools.partial(
        pl.kernel,
        out_type=jax.ShapeDtypeStruct((B, D), jnp.float32),
        mesh=mesh,
        scratch_types=[
            pltpu.VMEM((BW,), jnp.int32),
            pltpu.VMEM((NBUF, C, D), jnp.float32),
            pltpu.VMEM_SHARED((_NS, SBUF, C, D), jnp.float32),
            pltpu.SemaphoreType.DMA((NBUF,)),
            pltpu.SemaphoreType.DMA((SBUF,)),
            pltpu.SemaphoreType.DMA((max(SBUF, NBUF),)),
        ],
    )
    def k(idx_hbm, table_hbm, out_hbm, idx_v, bufs, shared, gsem, msem, ssem):
        wid = lax.axis_index("s") * _NC + lax.axis_index("c")
        sid = lax.axis_index("s")
        base = pl.multiple_of(wid * BW, 8)
        pltpu.sync_copy(idx_hbm.at[pl.ds(base, BW)], idx_v)

        def gather(j, s):
            off = pl.multiple_of(j * C, 8)
            pltpu.async_copy(
                table_hbm.at[idx_v.at[pl.ds(off, C)]], bufs.at[s], gsem.at[s]
            )

        def gather_wait(j, s):
            off = pl.multiple_of(j * C, 8)
            pltpu.make_async_copy(
                table_hbm.at[idx_v.at[pl.ds(off, C)]], bufs.at[s], gsem.at[s]
            ).wait()

        def move(s, m):
            pltpu.async_copy(bufs.at[s], shared.at[sid, m], msem.at[m])

        def move_wait(s, m):
            pltpu.make_async_copy(
                bufs.at[s], shared.at[sid, m], msem.at[m]
            ).wait()

        def store_direct(j, s):
            off = pl.multiple_of(j * C, 8)
            pltpu.async_copy(
                bufs.at[s], out_hbm.at[pl.ds(base + off, C)], ssem.at[s]
            )

        def store_direct_wait(j, s):
            off = pl.multiple_of(j * C, 8)
            pltpu.make_async_copy(
                bufs.at[s], out_hbm.at[pl.ds(base + off, C)], ssem.at[s]
            ).wait()

        def store_sp(j, m):
            off = pl.multiple_of(j * C, 8)
            pltpu.async_copy(
                shared.at[sid, m], out_hbm.at[pl.ds(base + off, C)], ssem.at[m]
            )

        def store_sp_wait(j, m):
            off = pl.multiple_of(j * C, 8)
            pltpu.make_async_copy(
                shared.at[sid, m], out_hbm.at[pl.ds(base + off, C)], ssem.at[m]
            ).wait()

        GD = 2

        @pl.when(sid % 2 == 0)
        def _():
            for b in range(GD):
                gather(b, b)

            def body(j, carry):
                s = lax.rem(j, NBUF)
                gather_wait(j, s)
                store_direct(j, s)
                sp = lax.rem(j + GD, NBUF)

                @pl.when(j + GD - NBUF >= 0)
                def _():
                    store_direct_wait(j + GD - NBUF, sp)

                @pl.when(j + GD < nchunk)
                def _():
                    gather(j + GD, sp)

                return carry

            lax.fori_loop(0, nchunk, body, 0)
            for jj in range(nchunk - (NBUF - GD), nchunk):
                store_direct_wait(jj, jj % NBUF)

        @pl.when(sid % 2 == 1)
        def _():
            for b in range(G):
                gather(b, b)

            def body(j, carry):
                s = lax.rem(j, NBUF)
                m = lax.rem(j, SBUF)
                gather_wait(j, s)

                @pl.when(j - SBUF >= 0)
                def _():
                    store_sp_wait(j - SBUF, m)

                move(s, m)

                @pl.when(j >= 1)
                def _():
                    mp = lax.rem(j - 1, SBUF)
                    move_wait(lax.rem(j - 1, NBUF), mp)
                    store_sp(j - 1, mp)

                @pl.when(j + G < nchunk)
                def _():
                    gather(j + G, lax.rem(j + G, NBUF))

                return carry

            lax.fori_loop(0, nchunk, body, 0)
            jl = nchunk - 1
            move_wait(jl % NBUF, jl % SBUF)
            store_sp(jl, jl % SBUF)
            for jj in range(nchunk - SBUF, nchunk):
                store_sp_wait(jj, jj % SBUF)

    return k(ids_flat, table)


def kernel(input_ids, word_embeddings):
    b, s = input_ids.shape
    ids_flat = input_ids.reshape(b * s).astype(jnp.int32)
    out = _emb_gather(ids_flat, word_embeddings)
    return out.reshape(b, s, word_embeddings.shape[1])


# speedup vs baseline: 1.0509x; 1.0509x over previous
import functools

import jax
import jax.numpy as jnp
from jax import lax
from jax.experimental import pallas as pl
from jax.experimental.pallas import tpu as pltpu
from jax.experimental.pallas import tpu_sc as plsc

_NC = 2
_NS = 16
_NW = _NC * _NS


def _emb_gather(ids_flat, table):
    B = ids_flat.shape[0]
    D = table.shape[1]
    BW = B // _NW
    C = 8
    NBUF = 4
    SBUF = 3
    G = 3
    nchunk = BW // C

    mesh = plsc.VectorSubcoreMesh(core_axis_name="c", subcore_axis_name="s")

    @functools.partial(
        pl.kernel,
        out_type=jax.ShapeDtypeStruct((B, D), jnp.float32),
        mesh=mesh,
        scratch_types=[
            pltpu.VMEM((BW,), jnp.int32),
            pltpu.VMEM((NBUF, C, D), jnp.float32),
            pltpu.VMEM_SHARED((_NS, SBUF, C, D), jnp.float32),
            pltpu.SemaphoreType.DMA((NBUF,)),
            pltpu.SemaphoreType.DMA((SBUF,)),
            pltpu.SemaphoreType.DMA((SBUF,)),
        ],
    )
    def k(idx_hbm, table_hbm, out_hbm, idx_v, bufs, shared, gsem, msem, ssem):
        wid = lax.axis_index("s") * _NC + lax.axis_index("c")
        sid = lax.axis_index("s")
        base = pl.multiple_of(wid * BW, 8)
        pltpu.sync_copy(idx_hbm.at[pl.ds(base, BW)], idx_v)

        def gather(j, s):
            off = pl.multiple_of(j * C, 8)
            pltpu.async_copy(
                table_hbm.at[idx_v.at[pl.ds(off, C)]], bufs.at[s], gsem.at[s]
            )

        def gather_wait(j, s):
            off = pl.multiple_of(j * C, 8)
            pltpu.make_async_copy(
                table_hbm.at[idx_v.at[pl.ds(off, C)]], bufs.at[s], gsem.at[s]
            ).wait()

        def move(s, m):
            pltpu.async_copy(bufs.at[s], shared.at[sid, m], msem.at[m])

        def move_wait(s, m):
            pltpu.make_async_copy(
                bufs.at[s], shared.at[sid, m], msem.at[m]
            ).wait()

        def store(j, m):
            off = pl.multiple_of(j * C, 8)
            pltpu.async_copy(
                shared.at[sid, m], out_hbm.at[pl.ds(base + off, C)], ssem.at[m]
            )

        def store_wait(j, m):
            off = pl.multiple_of(j * C, 8)
            pltpu.make_async_copy(
                shared.at[sid, m], out_hbm.at[pl.ds(base + off, C)], ssem.at[m]
            ).wait()

        for b in range(G):
            gather(b, b)

        def body(j, carry):
            s = lax.rem(j, NBUF)
            m = lax.rem(j, SBUF)
            gather_wait(j, s)

            @pl.when(j - SBUF >= 0)
            def _():
                store_wait(j - SBUF, m)

            move(s, m)

            @pl.when(j >= 1)
            def _():
                mp = lax.rem(j - 1, SBUF)
                move_wait(lax.rem(j - 1, NBUF), mp)
                store(j - 1, mp)

            @pl.when(j + G < nchunk)
            def _():
                gather(j + G, lax.rem(j + G, NBUF))

            return carry

        lax.fori_loop(0, nchunk, body, 0)
        jl = nchunk - 1
        move_wait(jl % NBUF, jl % SBUF)
        store(jl, jl % SBUF)
        for jj in range(nchunk - SBUF, nchunk):
            store_wait(jj, jj % SBUF)

    return k(ids_flat, table)


def kernel(input_ids, word_embeddings):
    b, s = input_ids.shape
    ids_flat = input_ids.reshape(b * s).astype(jnp.int32)
    out = _emb_gather(ids_flat, word_embeddings)
    return out.reshape(b, s, word_embeddings.shape[1])
